# Initial kernel scaffold; baseline (speedup 1.0000x reference)
import functools

import jax
import jax.numpy as jnp
from jax import lax
from jax.experimental import pallas as pl
from jax.experimental.pallas import tpu as pltpu
from jax.experimental.pallas import tpu_sc as plsc

N_NODES = 10000
N_PAD = 10240
N_EDGES = 320000
NW = 32
CH = 80
NCH = N_EDGES // NW // CH
ROWS_PER_SUB = N_PAD // 16

_MESH = plsc.VectorSubcoreMesh(core_axis_name="c", subcore_axis_name="s")


def _make_agg(D):
    @functools.partial(
        pl.kernel,
        out_type=jax.ShapeDtypeStruct((2, N_PAD, D), jnp.float32),
        mesh=_MESH,
        scratch_types=[
            pltpu.VMEM((NCH, CH), jnp.int32),
            pltpu.VMEM((NCH, CH), jnp.int32),
            pltpu.VMEM((CH, D), jnp.float32),
            pltpu.VMEM_SHARED((N_PAD, D), jnp.float32),
            pltpu.SemaphoreType.DMA,
        ],
    )
    def agg(y_hbm, src_hbm, dst_hbm, out_hbm, src_v, dst_v, rows_v, acc_sh, sem):
        cid = lax.axis_index("c")
        sid = lax.axis_index("s")
        wid = sid * 2 + cid

        @pl.loop(0, CH)
        def _(i):
            @pl.loop(0, D // 16)
            def _(k):
                rows_v[i, pl.ds(k * 16, 16)] = jnp.zeros((16,), jnp.float32)

        @pl.loop(0, ROWS_PER_SUB // CH)
        def _(r):
            pltpu.sync_copy(rows_v, acc_sh.at[pl.ds(sid * ROWS_PER_SUB + r * CH, CH)])

        plsc.subcore_barrier()

        pltpu.sync_copy(src_hbm.at[pl.ds(wid * NCH, NCH)], src_v)
        pltpu.sync_copy(dst_hbm.at[pl.ds(wid * NCH, NCH)], dst_v)

        @pl.loop(0, NCH)
        def _(j):
            pltpu.async_copy(y_hbm.at[src_v.at[j]], rows_v, sem).wait()
            pltpu.sync_copy(rows_v, acc_sh.at[dst_v.at[j]], add=True)

        plsc.subcore_barrier()
        pltpu.sync_copy(
            acc_sh.at[pl.ds(sid * ROWS_PER_SUB, ROWS_PER_SUB)],
            out_hbm.at[cid, pl.ds(sid * ROWS_PER_SUB, ROWS_PER_SUB)],
        )

    return agg


_agg128 = _make_agg(128)
_agg64 = _make_agg(64)


@functools.partial(
    pl.kernel,
    out_type=jax.ShapeDtypeStruct((2, N_PAD, 16), jnp.float32),
    mesh=_MESH,
    scratch_types=[
        pltpu.VMEM((NCH, CH), jnp.int32),
        pltpu.VMEM((CH, 16), jnp.float32),
        pltpu.VMEM((CH, 16), jnp.float32),
        pltpu.VMEM_SHARED((N_PAD, 16), jnp.float32),
    ],
)
def _deg_kernel(dst_hbm, out_hbm, dst_v, ones_v, zeros_v, acc_sh):
    cid = lax.axis_index("c")
    sid = lax.axis_index("s")
    wid = sid * 2 + cid

    @pl.loop(0, CH)
    def _(i):
        ones_v[i, pl.ds(0, 16)] = jnp.ones((16,), jnp.float32)
        zeros_v[i, pl.ds(0, 16)] = jnp.zeros((16,), jnp.float32)

    @pl.loop(0, ROWS_PER_SUB // CH)
    def _(r):
        pltpu.sync_copy(zeros_v, acc_sh.at[pl.ds(sid * ROWS_PER_SUB + r * CH, CH)])

    plsc.subcore_barrier()
    pltpu.sync_copy(dst_hbm.at[pl.ds(wid * NCH, NCH)], dst_v)

    @pl.loop(0, NCH)
    def _(j):
        pltpu.sync_copy(ones_v, acc_sh.at[dst_v.at[j]], add=True)

    plsc.subcore_barrier()
    pltpu.sync_copy(
        acc_sh.at[pl.ds(sid * ROWS_PER_SUB, ROWS_PER_SUB)],
        out_hbm.at[cid, pl.ds(sid * ROWS_PER_SUB, ROWS_PER_SUB)],
    )


_BS = 2048


def _proj(x, w):
    D = w.shape[0]

    def body(x_ref, w_ref, o_ref):
        o_ref[...] = lax.dot_general(
            x_ref[...], w_ref[...], (((1,), (1,)), ((), ())),
            preferred_element_type=jnp.float32,
        )

    return pl.pallas_call(
        body,
        grid=(N_PAD // _BS,),
        in_specs=[
            pl.BlockSpec((_BS, 128), lambda i: (i, 0)),
            pl.BlockSpec((D, 128), lambda i: (0, 0)),
        ],
        out_specs=pl.BlockSpec((_BS, D), lambda i: (i, 0)),
        out_shape=jax.ShapeDtypeStruct((N_PAD, D), jnp.float32),
    )(x, w)


def _combine(x, w_self, b, aggp, degp):
    D = w_self.shape[0]

    def body(x_ref, w_ref, b_ref, a_ref, d_ref, o_ref):
        z = lax.dot_general(
            x_ref[...], w_ref[...], (((1,), (1,)), ((), ())),
            preferred_element_type=jnp.float32,
        )
        deg = d_ref[0, :, 0:1] + d_ref[1, :, 0:1]
        inv = 1.0 / jnp.maximum(deg, 1.0)
        agg = a_ref[0] + a_ref[1]
        o_ref[...] = jax.nn.sigmoid(z + b_ref[...] + agg * inv)

    return pl.pallas_call(
        body,
        grid=(N_PAD // _BS,),
        in_specs=[
            pl.BlockSpec((_BS, 128), lambda i: (i, 0)),
            pl.BlockSpec((D, 128), lambda i: (0, 0)),
            pl.BlockSpec((1, D), lambda i: (0, 0)),
            pl.BlockSpec((2, _BS, D), lambda i: (0, i, 0)),
            pl.BlockSpec((2, _BS, 16), lambda i: (0, i, 0)),
        ],
        out_specs=pl.BlockSpec((_BS, D), lambda i: (i, 0)),
        out_shape=jax.ShapeDtypeStruct((N_PAD, D), jnp.float32),
    )(x, w_self, b.reshape(1, D), aggp, degp)


def kernel(inputs, W1_self, W1_neigh, b1, W2_self, W2_neigh, b2,
           W3_self, W3_neigh, b3, edge_index):
    x = jnp.pad(inputs, ((0, N_PAD - N_NODES), (0, 0)))
    srcm = edge_index[0].reshape(NW * NCH, CH)
    dstm = edge_index[1].reshape(NW * NCH, CH)

    degp = _deg_kernel(dstm)

    y1 = _proj(x, W1_neigh)
    a1 = _agg128(y1, srcm, dstm)
    h1 = _combine(x, W1_self, b1, a1, degp)

    y2 = _proj(h1, W2_neigh)
    a2 = _agg128(y2, srcm, dstm)
    h2 = _combine(h1, W2_self, b2, a2, degp)

    y3 = _proj(h2, W3_neigh)
    a3 = _agg64(y3, srcm, dstm)
    h3 = _combine(h2, W3_self, b3, a3, degp)

    return h3[:N_NODES]


# speedup vs baseline: 3.0373x; 3.0373x over previous
import functools

import jax
import jax.numpy as jnp
from jax import lax
from jax.experimental import pallas as pl
from jax.experimental.pallas import tpu as pltpu
from jax.experimental.pallas import tpu_sc as plsc

N_NODES = 10000
N_PAD = 10240
N_EDGES = 320000
F = 128
NW = 32
CH = 80
NCH_REAL = N_EDGES // NW // CH
NCH = 128
ROWS_PER_SUB = N_PAD // 16

_MESH = plsc.VectorSubcoreMesh(core_axis_name="c", subcore_axis_name="s")


@functools.partial(
    pl.kernel,
    out_type=jax.ShapeDtypeStruct((2, N_PAD, F), jnp.float32),
    mesh=_MESH,
    scratch_types=[
        pltpu.VMEM((NCH, CH), jnp.int32),
        pltpu.VMEM((NCH, CH), jnp.int32),
        pltpu.VMEM((CH, F), jnp.float32),
        pltpu.VMEM_SHARED((N_PAD, F), jnp.float32),
        pltpu.SemaphoreType.DMA,
    ],
)
def _agg(x_hbm, src_hbm, dst_hbm, out_hbm, src_v, dst_v, rows_v, acc_sh, sem):
    cid = lax.axis_index("c")
    sid = lax.axis_index("s")
    wid = sid * 2 + cid

    @pl.loop(0, CH)
    def _(i):
        @pl.loop(0, F // 16)
        def _(k):
            rows_v[i, pl.ds(k * 16, 16)] = jnp.zeros((16,), jnp.float32)

    @pl.loop(0, ROWS_PER_SUB // CH)
    def _(r):
        pltpu.sync_copy(rows_v, acc_sh.at[pl.ds(sid * ROWS_PER_SUB + r * CH, CH)])

    plsc.subcore_barrier()

    pltpu.sync_copy(src_hbm.at[pl.ds(wid * NCH, NCH)], src_v)
    pltpu.sync_copy(dst_hbm.at[pl.ds(wid * NCH, NCH)], dst_v)

    @pl.loop(0, NCH)
    def _(j):
        pltpu.async_copy(x_hbm.at[src_v.at[j]], rows_v, sem).wait()
        pltpu.sync_copy(rows_v, acc_sh.at[dst_v.at[j]], add=True)

    plsc.subcore_barrier()
    pltpu.sync_copy(
        acc_sh.at[pl.ds(sid * ROWS_PER_SUB, ROWS_PER_SUB)],
        out_hbm.at[cid, pl.ds(sid * ROWS_PER_SUB, ROWS_PER_SUB)],
    )


@functools.partial(
    pl.kernel,
    out_type=jax.ShapeDtypeStruct((2, N_PAD, F), jnp.float32),
    mesh=_MESH,
    scratch_types=[
        pltpu.VMEM((NCH, CH), jnp.int32),
        pltpu.VMEM((CH, F), jnp.float32),
        pltpu.VMEM((CH, F), jnp.float32),
        pltpu.VMEM_SHARED((N_PAD, F), jnp.float32),
    ],
)
def _deg_kernel(dst_hbm, out_hbm, dst_v, ones_v, zeros_v, acc_sh):
    cid = lax.axis_index("c")
    sid = lax.axis_index("s")
    wid = sid * 2 + cid

    @pl.loop(0, CH)
    def _(i):
        @pl.loop(0, F // 16)
        def _(k):
            ones_v[i, pl.ds(k * 16, 16)] = jnp.ones((16,), jnp.float32)
            zeros_v[i, pl.ds(k * 16, 16)] = jnp.zeros((16,), jnp.float32)

    @pl.loop(0, ROWS_PER_SUB // CH)
    def _(r):
        pltpu.sync_copy(zeros_v, acc_sh.at[pl.ds(sid * ROWS_PER_SUB + r * CH, CH)])

    plsc.subcore_barrier()
    pltpu.sync_copy(dst_hbm.at[pl.ds(wid * NCH, NCH)], dst_v)

    @pl.loop(0, NCH)
    def _(j):
        pltpu.sync_copy(ones_v, acc_sh.at[dst_v.at[j]], add=True)

    plsc.subcore_barrier()
    pltpu.sync_copy(
        acc_sh.at[pl.ds(sid * ROWS_PER_SUB, ROWS_PER_SUB)],
        out_hbm.at[cid, pl.ds(sid * ROWS_PER_SUB, ROWS_PER_SUB)],
    )


_BS = 2048


def _self_proj(x, w, b):
    D = w.shape[0]

    def body(x_ref, w_ref, b_ref, o_ref):
        o_ref[...] = lax.dot_general(
            x_ref[...], w_ref[...], (((1,), (1,)), ((), ())),
            preferred_element_type=jnp.float32,
        ) + b_ref[...]

    return pl.pallas_call(
        body,
        grid=(N_PAD // _BS,),
        in_specs=[
            pl.BlockSpec((_BS, F), lambda i: (i, 0)),
            pl.BlockSpec((D, F), lambda i: (0, 0)),
            pl.BlockSpec((1, D), lambda i: (0, 0)),
        ],
        out_specs=pl.BlockSpec((_BS, D), lambda i: (i, 0)),
        out_shape=jax.ShapeDtypeStruct((N_PAD, D), jnp.float32),
    )(x, w, b.reshape(1, D))


def _combine(z, w_neigh, aggp, degp):
    D = w_neigh.shape[0]

    def body(z_ref, w_ref, a_ref, d_ref, o_ref):
        deg = d_ref[0, :, 0:1] + d_ref[1, :, 0:1]
        inv = 1.0 / jnp.maximum(deg, 1.0)
        mean = (a_ref[0] + a_ref[1]) * inv
        mn = lax.dot_general(
            mean, w_ref[...], (((1,), (1,)), ((), ())),
            preferred_element_type=jnp.float32,
        )
        o_ref[...] = jax.nn.sigmoid(z_ref[...] + mn)

    return pl.pallas_call(
        body,
        grid=(N_PAD // _BS,),
        in_specs=[
            pl.BlockSpec((_BS, D), lambda i: (i, 0)),
            pl.BlockSpec((D, F), lambda i: (0, 0)),
            pl.BlockSpec((2, _BS, F), lambda i: (0, i, 0)),
            pl.BlockSpec((2, _BS, F), lambda i: (0, i, 0)),
        ],
        out_specs=pl.BlockSpec((_BS, D), lambda i: (i, 0)),
        out_shape=jax.ShapeDtypeStruct((N_PAD, D), jnp.float32),
    )(z, w_neigh, aggp, degp)


def kernel(inputs, W1_self, W1_neigh, b1, W2_self, W2_neigh, b2,
           W3_self, W3_neigh, b3, edge_index):
    x = jnp.pad(inputs, ((0, N_PAD - N_NODES), (0, 0)))
    pad3 = ((0, 0), (0, NCH - NCH_REAL), (0, 0))
    srcm = jnp.pad(edge_index[0].reshape(NW, NCH_REAL, CH), pad3,
                   constant_values=N_NODES).reshape(NW * NCH, CH)
    dstm = jnp.pad(edge_index[1].reshape(NW, NCH_REAL, CH), pad3,
                   constant_values=N_NODES).reshape(NW * NCH, CH)

    degp = _deg_kernel(dstm)

    h = x
    for w_self, w_neigh, b in ((W1_self, W1_neigh, b1),
                               (W2_self, W2_neigh, b2),
                               (W3_self, W3_neigh, b3)):
        aggp = _agg(h, srcm, dstm)
        z = _self_proj(h, w_self, b)
        h = _combine(z, w_neigh, aggp, degp)

    return h[:N_NODES]


# speedup vs baseline: 3.3058x; 1.0884x over previous
import functools

import jax
import jax.numpy as jnp
from jax import lax
from jax.experimental import pallas as pl
from jax.experimental.pallas import tpu as pltpu
from jax.experimental.pallas import tpu_sc as plsc

N_NODES = 10000
N_PAD = 10240
N_EDGES = 320000
F = 128
NW = 32
CH = 128
EPW = N_EDGES // NW
NCH = 80
SUP = 8
NBUF = 2
ROWS_PER_SUB = N_PAD // 16

_MESH = plsc.VectorSubcoreMesh(core_axis_name="c", subcore_axis_name="s")


@functools.partial(
    pl.kernel,
    out_type=jax.ShapeDtypeStruct((2, N_PAD, F), jnp.float32),
    mesh=_MESH,
    scratch_types=[
        pltpu.VMEM((SUP, CH), jnp.int32),
        pltpu.VMEM((SUP, CH), jnp.int32),
        pltpu.VMEM((CH, F), jnp.float32),
        pltpu.VMEM((CH, F), jnp.float32),
        pltpu.VMEM_SHARED((N_PAD, F), jnp.float32),
        pltpu.SemaphoreType.DMA,
        pltpu.SemaphoreType.DMA,
    ],
)
def _agg(x_hbm, src_hbm, dst_hbm, out_hbm, src_v, dst_v, buf0, buf1, acc_sh,
         sem0, sem1):
    bufs = (buf0, buf1)
    sems = (sem0, sem1)
    cid = lax.axis_index("c")
    sid = lax.axis_index("s")
    wid = sid * 2 + cid

    @pl.loop(0, CH)
    def _(i):
        @pl.loop(0, F // 16)
        def _(k):
            buf0[i, pl.ds(k * 16, 16)] = jnp.zeros((16,), jnp.float32)

    @pl.loop(0, ROWS_PER_SUB // CH)
    def _(r):
        pltpu.sync_copy(buf0, acc_sh.at[pl.ds(sid * ROWS_PER_SUB + r * CH, CH)])

    plsc.subcore_barrier()

    @pl.loop(0, NCH // SUP)
    def _(s):
        base = wid * NCH + s * SUP
        pltpu.sync_copy(src_hbm.at[pl.ds(base, SUP)], src_v)
        pltpu.sync_copy(dst_hbm.at[pl.ds(base, SUP)], dst_v)

        @pl.loop(0, SUP // NBUF)
        def _(u):
            descs = [
                pltpu.async_copy(
                    x_hbm.at[src_v.at[u * NBUF + b]], bufs[b], sems[b])
                for b in range(NBUF)
            ]
            for b in range(NBUF):
                descs[b].wait()
                pltpu.sync_copy(bufs[b], acc_sh.at[dst_v.at[u * NBUF + b]],
                                add=True)

    plsc.subcore_barrier()
    pltpu.sync_copy(
        acc_sh.at[pl.ds(sid * ROWS_PER_SUB, ROWS_PER_SUB)],
        out_hbm.at[cid, pl.ds(sid * ROWS_PER_SUB, ROWS_PER_SUB)],
    )


@functools.partial(
    pl.kernel,
    out_type=jax.ShapeDtypeStruct((2, N_PAD, F), jnp.float32),
    mesh=_MESH,
    scratch_types=[
        pltpu.VMEM((NCH, CH), jnp.int32),
        pltpu.VMEM((CH, F), jnp.float32),
        pltpu.VMEM_SHARED((N_PAD, F), jnp.float32),
    ],
)
def _deg_kernel(dst_hbm, out_hbm, dst_v, ones_v, acc_sh):
    cid = lax.axis_index("c")
    sid = lax.axis_index("s")
    wid = sid * 2 + cid

    @pl.loop(0, CH)
    def _(i):
        @pl.loop(0, F // 16)
        def _(k):
            ones_v[i, pl.ds(k * 16, 16)] = jnp.zeros((16,), jnp.float32)

    @pl.loop(0, ROWS_PER_SUB // CH)
    def _(r):
        pltpu.sync_copy(ones_v, acc_sh.at[pl.ds(sid * ROWS_PER_SUB + r * CH, CH)])

    @pl.loop(0, CH)
    def _(i):
        @pl.loop(0, F // 16)
        def _(k):
            ones_v[i, pl.ds(k * 16, 16)] = jnp.ones((16,), jnp.float32)

    plsc.subcore_barrier()
    pltpu.sync_copy(dst_hbm.at[pl.ds(wid * NCH, NCH)], dst_v)

    @pl.loop(0, NCH)
    def _(j):
        pltpu.sync_copy(ones_v, acc_sh.at[dst_v.at[j]], add=True)

    plsc.subcore_barrier()
    pltpu.sync_copy(
        acc_sh.at[pl.ds(sid * ROWS_PER_SUB, ROWS_PER_SUB)],
        out_hbm.at[cid, pl.ds(sid * ROWS_PER_SUB, ROWS_PER_SUB)],
    )


_BS = 2048


def _self_proj(x, w, b):
    D = w.shape[0]

    def body(x_ref, w_ref, b_ref, o_ref):
        o_ref[...] = lax.dot_general(
            x_ref[...], w_ref[...], (((1,), (1,)), ((), ())),
            preferred_element_type=jnp.float32,
        ) + b_ref[...]

    return pl.pallas_call(
        body,
        grid=(N_PAD // _BS,),
        in_specs=[
            pl.BlockSpec((_BS, F), lambda i: (i, 0)),
            pl.BlockSpec((D, F), lambda i: (0, 0)),
            pl.BlockSpec((1, D), lambda i: (0, 0)),
        ],
        out_specs=pl.BlockSpec((_BS, D), lambda i: (i, 0)),
        out_shape=jax.ShapeDtypeStruct((N_PAD, D), jnp.float32),
    )(x, w, b.reshape(1, D))


def _combine(z, w_neigh, aggp, degp):
    D = w_neigh.shape[0]

    def body(z_ref, w_ref, a_ref, d_ref, o_ref):
        deg = d_ref[0, :, 0:1] + d_ref[1, :, 0:1]
        inv = 1.0 / jnp.maximum(deg, 1.0)
        mean = (a_ref[0] + a_ref[1]) * inv
        mn = lax.dot_general(
            mean, w_ref[...], (((1,), (1,)), ((), ())),
            preferred_element_type=jnp.float32,
        )
        o_ref[...] = jax.nn.sigmoid(z_ref[...] + mn)

    return pl.pallas_call(
        body,
        grid=(N_PAD // _BS,),
        in_specs=[
            pl.BlockSpec((_BS, D), lambda i: (i, 0)),
            pl.BlockSpec((D, F), lambda i: (0, 0)),
            pl.BlockSpec((2, _BS, F), lambda i: (0, i, 0)),
            pl.BlockSpec((2, _BS, F), lambda i: (0, i, 0)),
        ],
        out_specs=pl.BlockSpec((_BS, D), lambda i: (i, 0)),
        out_shape=jax.ShapeDtypeStruct((N_PAD, D), jnp.float32),
    )(z, w_neigh, aggp, degp)


def kernel(inputs, W1_self, W1_neigh, b1, W2_self, W2_neigh, b2,
           W3_self, W3_neigh, b3, edge_index):
    x = jnp.pad(inputs, ((0, N_PAD - N_NODES), (0, 0)))
    pad2 = ((0, 0), (0, NCH * CH - EPW))
    srcm = jnp.pad(edge_index[0].reshape(NW, EPW), pad2,
                   constant_values=N_NODES).reshape(NW * NCH, CH)
    dstm = jnp.pad(edge_index[1].reshape(NW, EPW), pad2,
                   constant_values=N_NODES).reshape(NW * NCH, CH)

    degp = _deg_kernel(dstm)

    h = x
    for w_self, w_neigh, b in ((W1_self, W1_neigh, b1),
                               (W2_self, W2_neigh, b2),
                               (W3_self, W3_neigh, b3)):
        aggp = _agg(h, srcm, dstm)
        z = _self_proj(h, w_self, b)
        h = _combine(z, w_neigh, aggp, degp)

    return h[:N_NODES]


# speedup vs baseline: 3.6864x; 1.1151x over previous
import functools

import jax
import jax.numpy as jnp
from jax import lax
from jax.experimental import pallas as pl
from jax.experimental.pallas import tpu as pltpu
from jax.experimental.pallas import tpu_sc as plsc

N_NODES = 10000
N_PAD = 10240
N_EDGES = 320000
F = 128
NW = 32
CH = 128
EPW = N_EDGES // NW
NCH = 80
SUP = 8
NBUF = 2
ROWS_PER_SUB = N_PAD // 16

_MESH = plsc.VectorSubcoreMesh(core_axis_name="c", subcore_axis_name="s")


NGRP = NCH // SUP


@functools.partial(
    pl.kernel,
    out_type=jax.ShapeDtypeStruct((2, N_PAD, F), jnp.float32),
    mesh=_MESH,
    scratch_types=[
        pltpu.VMEM((SUP, CH), jnp.int32),
        pltpu.VMEM((SUP, CH), jnp.int32),
        pltpu.VMEM((SUP, CH), jnp.int32),
        pltpu.VMEM((SUP, CH), jnp.int32),
        pltpu.VMEM((CH, F), jnp.float32),
        pltpu.VMEM((CH, F), jnp.float32),
        pltpu.VMEM_SHARED((N_PAD, F), jnp.float32),
        pltpu.SemaphoreType.DMA,
        pltpu.SemaphoreType.DMA,
        pltpu.SemaphoreType.DMA,
    ],
)
def _agg(x_hbm, src_hbm, dst_hbm, out_hbm, srcA, dstA, srcB, dstB,
         buf0, buf1, acc_sh, gsem0, gsem1, isem):
    bufs = (buf0, buf1)
    gsems = (gsem0, gsem1)
    idxs = ((srcA, dstA), (srcB, dstB))
    cid = lax.axis_index("c")
    sid = lax.axis_index("s")
    wid = sid * 2 + cid

    @pl.loop(0, CH)
    def _(i):
        @pl.loop(0, F // 16)
        def _(k):
            buf0[i, pl.ds(k * 16, 16)] = jnp.zeros((16,), jnp.float32)

    @pl.loop(0, ROWS_PER_SUB // CH)
    def _(r):
        pltpu.sync_copy(buf0, acc_sh.at[pl.ds(sid * ROWS_PER_SUB + r * CH, CH)])

    plsc.subcore_barrier()

    def wait_gather(b):
        pltpu.make_async_copy(x_hbm.at[srcA.at[0]], bufs[b], gsems[b]).wait()

    def stage(s, par):
        base = wid * NCH + s * SUP
        pltpu.async_copy(src_hbm.at[pl.ds(base, SUP)], idxs[par][0], isem)
        pltpu.async_copy(dst_hbm.at[pl.ds(base, SUP)], idxs[par][1], isem)

    def wait_stage():
        pltpu.make_async_copy(src_hbm.at[pl.ds(0, SUP)], srcA, isem).wait()
        pltpu.make_async_copy(src_hbm.at[pl.ds(0, SUP)], srcB, isem).wait()

    stage(0, 0)
    wait_stage()
    pltpu.async_copy(x_hbm.at[srcA.at[0]], buf0, gsem0)
    pltpu.async_copy(x_hbm.at[srcA.at[1]], buf1, gsem1)

    @pl.loop(0, NGRP // 2)
    def _(g2):
        for par in range(2):
            s = g2 * 2 + par
            cur_s, cur_d = idxs[par]
            nxt_s, _ = idxs[1 - par]

            @pl.when(s + 1 < NGRP)
            def _():
                stage(s + 1, 1 - par)

            for u in range(SUP):
                b = u % 2
                wait_gather(b)
                pltpu.sync_copy(bufs[b], acc_sh.at[cur_d.at[u]], add=True)
                if u == SUP - 2:
                    @pl.when(s + 1 < NGRP)
                    def _():
                        wait_stage()
                nxt_row = cur_s.at[u + 2] if u + 2 < SUP else nxt_s.at[u + 2 - SUP]

                @pl.when(s * SUP + u + 2 < NCH)
                def _():
                    pltpu.async_copy(x_hbm.at[nxt_row], bufs[b], gsems[b])

    plsc.subcore_barrier()
    pltpu.sync_copy(
        acc_sh.at[pl.ds(sid * ROWS_PER_SUB, ROWS_PER_SUB)],
        out_hbm.at[cid, pl.ds(sid * ROWS_PER_SUB, ROWS_PER_SUB)],
    )


@functools.partial(
    pl.kernel,
    out_type=jax.ShapeDtypeStruct((2, N_PAD, F), jnp.float32),
    mesh=_MESH,
    scratch_types=[
        pltpu.VMEM((NCH, CH), jnp.int32),
        pltpu.VMEM((CH, F), jnp.float32),
        pltpu.VMEM_SHARED((N_PAD, F), jnp.float32),
    ],
)
def _deg_kernel(dst_hbm, out_hbm, dst_v, ones_v, acc_sh):
    cid = lax.axis_index("c")
    sid = lax.axis_index("s")
    wid = sid * 2 + cid

    @pl.loop(0, CH)
    def _(i):
        @pl.loop(0, F // 16)
        def _(k):
            ones_v[i, pl.ds(k * 16, 16)] = jnp.zeros((16,), jnp.float32)

    @pl.loop(0, ROWS_PER_SUB // CH)
    def _(r):
        pltpu.sync_copy(ones_v, acc_sh.at[pl.ds(sid * ROWS_PER_SUB + r * CH, CH)])

    @pl.loop(0, CH)
    def _(i):
        @pl.loop(0, F // 16)
        def _(k):
            ones_v[i, pl.ds(k * 16, 16)] = jnp.ones((16,), jnp.float32)

    plsc.subcore_barrier()
    pltpu.sync_copy(dst_hbm.at[pl.ds(wid * NCH, NCH)], dst_v)

    @pl.loop(0, NCH)
    def _(j):
        pltpu.sync_copy(ones_v, acc_sh.at[dst_v.at[j]], add=True)

    plsc.subcore_barrier()
    pltpu.sync_copy(
        acc_sh.at[pl.ds(sid * ROWS_PER_SUB, ROWS_PER_SUB)],
        out_hbm.at[cid, pl.ds(sid * ROWS_PER_SUB, ROWS_PER_SUB)],
    )


_BS = 2048


def _self_proj(x, w, b):
    D = w.shape[0]

    def body(x_ref, w_ref, b_ref, o_ref):
        o_ref[...] = lax.dot_general(
            x_ref[...], w_ref[...], (((1,), (1,)), ((), ())),
            preferred_element_type=jnp.float32,
        ) + b_ref[...]

    return pl.pallas_call(
        body,
        grid=(N_PAD // _BS,),
        in_specs=[
            pl.BlockSpec((_BS, F), lambda i: (i, 0)),
            pl.BlockSpec((D, F), lambda i: (0, 0)),
            pl.BlockSpec((1, D), lambda i: (0, 0)),
        ],
        out_specs=pl.BlockSpec((_BS, D), lambda i: (i, 0)),
        out_shape=jax.ShapeDtypeStruct((N_PAD, D), jnp.float32),
    )(x, w, b.reshape(1, D))


def _combine(z, w_neigh, aggp, degp):
    D = w_neigh.shape[0]

    def body(z_ref, w_ref, a_ref, d_ref, o_ref):
        deg = d_ref[0, :, 0:1] + d_ref[1, :, 0:1]
        inv = 1.0 / jnp.maximum(deg, 1.0)
        mean = (a_ref[0] + a_ref[1]) * inv
        mn = lax.dot_general(
            mean, w_ref[...], (((1,), (1,)), ((), ())),
            preferred_element_type=jnp.float32,
        )
        o_ref[...] = jax.nn.sigmoid(z_ref[...] + mn)

    return pl.pallas_call(
        body,
        grid=(N_PAD // _BS,),
        in_specs=[
            pl.BlockSpec((_BS, D), lambda i: (i, 0)),
            pl.BlockSpec((D, F), lambda i: (0, 0)),
            pl.BlockSpec((2, _BS, F), lambda i: (0, i, 0)),
            pl.BlockSpec((2, _BS, F), lambda i: (0, i, 0)),
        ],
        out_specs=pl.BlockSpec((_BS, D), lambda i: (i, 0)),
        out_shape=jax.ShapeDtypeStruct((N_PAD, D), jnp.float32),
    )(z, w_neigh, aggp, degp)


def kernel(inputs, W1_self, W1_neigh, b1, W2_self, W2_neigh, b2,
           W3_self, W3_neigh, b3, edge_index):
    x = jnp.pad(inputs, ((0, N_PAD - N_NODES), (0, 0)))
    pad2 = ((0, 0), (0, NCH * CH - EPW))
    srcm = jnp.pad(edge_index[0].reshape(NW, EPW), pad2,
                   constant_values=N_NODES).reshape(NW * NCH, CH)
    dstm = jnp.pad(edge_index[1].reshape(NW, EPW), pad2,
                   constant_values=N_NODES).reshape(NW * NCH, CH)

    degp = _deg_kernel(dstm)

    h = x
    for w_self, w_neigh, b in ((W1_self, W1_neigh, b1),
                               (W2_self, W2_neigh, b2),
                               (W3_self, W3_neigh, b3)):
        aggp = _agg(h, srcm, dstm)
        z = _self_proj(h, w_self, b)
        h = _combine(z, w_neigh, aggp, degp)

    return h[:N_NODES]


# speedup vs baseline: 3.7736x; 1.0237x over previous
import functools

import jax
import jax.numpy as jnp
from jax import lax
from jax.experimental import pallas as pl
from jax.experimental.pallas import tpu as pltpu
from jax.experimental.pallas import tpu_sc as plsc

N_NODES = 10000
N_PAD = 10240
N_EDGES = 320000
F = 128
NW = 32
CH = 64
EPW = N_EDGES // NW
NCH = 160
SUP = 8
NBUF = 4
ROWS_PER_SUB = N_PAD // 16

_MESH = plsc.VectorSubcoreMesh(core_axis_name="c", subcore_axis_name="s")


NGRP = NCH // SUP


@functools.partial(
    pl.kernel,
    out_type=jax.ShapeDtypeStruct((2, N_PAD, F), jnp.float32),
    mesh=_MESH,
    scratch_types=[
        pltpu.VMEM((SUP, CH), jnp.int32),
        pltpu.VMEM((SUP, CH), jnp.int32),
        pltpu.VMEM((SUP, CH), jnp.int32),
        pltpu.VMEM((SUP, CH), jnp.int32),
        pltpu.VMEM((CH, F), jnp.float32),
        pltpu.VMEM((CH, F), jnp.float32),
        pltpu.VMEM((CH, F), jnp.float32),
        pltpu.VMEM((CH, F), jnp.float32),
        pltpu.VMEM_SHARED((N_PAD, F), jnp.float32),
        pltpu.SemaphoreType.DMA,
        pltpu.SemaphoreType.DMA,
        pltpu.SemaphoreType.DMA,
        pltpu.SemaphoreType.DMA,
        pltpu.SemaphoreType.DMA,
    ],
)
def _agg(x_hbm, src_hbm, dst_hbm, out_hbm, srcA, dstA, srcB, dstB,
         buf0, buf1, buf2, buf3, acc_sh, gsem0, gsem1, gsem2, gsem3, isem):
    bufs = (buf0, buf1, buf2, buf3)
    gsems = (gsem0, gsem1, gsem2, gsem3)
    idxs = ((srcA, dstA), (srcB, dstB))
    cid = lax.axis_index("c")
    sid = lax.axis_index("s")
    wid = sid * 2 + cid

    @pl.loop(0, CH)
    def _(i):
        @pl.loop(0, F // 16)
        def _(k):
            buf0[i, pl.ds(k * 16, 16)] = jnp.zeros((16,), jnp.float32)

    @pl.loop(0, ROWS_PER_SUB // CH)
    def _(r):
        pltpu.sync_copy(buf0, acc_sh.at[pl.ds(sid * ROWS_PER_SUB + r * CH, CH)])

    plsc.subcore_barrier()

    def wait_gather(b):
        pltpu.make_async_copy(x_hbm.at[srcA.at[0]], bufs[b], gsems[b]).wait()

    def stage(s, par):
        base = wid * NCH + s * SUP
        pltpu.async_copy(src_hbm.at[pl.ds(base, SUP)], idxs[par][0], isem)
        pltpu.async_copy(dst_hbm.at[pl.ds(base, SUP)], idxs[par][1], isem)

    def wait_stage():
        pltpu.make_async_copy(src_hbm.at[pl.ds(0, SUP)], srcA, isem).wait()
        pltpu.make_async_copy(src_hbm.at[pl.ds(0, SUP)], srcB, isem).wait()

    stage(0, 0)
    wait_stage()
    for b in range(NBUF):
        pltpu.async_copy(x_hbm.at[srcA.at[b]], bufs[b], gsems[b])

    @pl.loop(0, NGRP // 2)
    def _(g2):
        for par in range(2):
            s = g2 * 2 + par
            cur_s, cur_d = idxs[par]
            nxt_s, _ = idxs[1 - par]

            @pl.when(s + 1 < NGRP)
            def _():
                stage(s + 1, 1 - par)

            for u in range(SUP):
                b = u % NBUF
                wait_gather(b)
                pltpu.sync_copy(bufs[b], acc_sh.at[cur_d.at[u]], add=True)
                if u == SUP - NBUF:
                    @pl.when(s + 1 < NGRP)
                    def _():
                        wait_stage()
                nxt_row = (cur_s.at[u + NBUF] if u + NBUF < SUP
                           else nxt_s.at[u + NBUF - SUP])

                @pl.when(s * SUP + u + NBUF < NCH)
                def _():
                    pltpu.async_copy(x_hbm.at[nxt_row], bufs[b], gsems[b])

    plsc.subcore_barrier()
    pltpu.sync_copy(
        acc_sh.at[pl.ds(sid * ROWS_PER_SUB, ROWS_PER_SUB)],
        out_hbm.at[cid, pl.ds(sid * ROWS_PER_SUB, ROWS_PER_SUB)],
    )


@functools.partial(
    pl.kernel,
    out_type=jax.ShapeDtypeStruct((2, N_PAD, F), jnp.float32),
    mesh=_MESH,
    scratch_types=[
        pltpu.VMEM((NCH, CH), jnp.int32),
        pltpu.VMEM((CH, F), jnp.float32),
        pltpu.VMEM_SHARED((N_PAD, F), jnp.float32),
    ],
)
def _deg_kernel(dst_hbm, out_hbm, dst_v, ones_v, acc_sh):
    cid = lax.axis_index("c")
    sid = lax.axis_index("s")
    wid = sid * 2 + cid

    @pl.loop(0, CH)
    def _(i):
        @pl.loop(0, F // 16)
        def _(k):
            ones_v[i, pl.ds(k * 16, 16)] = jnp.zeros((16,), jnp.float32)

    @pl.loop(0, ROWS_PER_SUB // CH)
    def _(r):
        pltpu.sync_copy(ones_v, acc_sh.at[pl.ds(sid * ROWS_PER_SUB + r * CH, CH)])

    @pl.loop(0, CH)
    def _(i):
        @pl.loop(0, F // 16)
        def _(k):
            ones_v[i, pl.ds(k * 16, 16)] = jnp.ones((16,), jnp.float32)

    plsc.subcore_barrier()
    pltpu.sync_copy(dst_hbm.at[pl.ds(wid * NCH, NCH)], dst_v)

    @pl.loop(0, NCH)
    def _(j):
        pltpu.sync_copy(ones_v, acc_sh.at[dst_v.at[j]], add=True)

    plsc.subcore_barrier()
    pltpu.sync_copy(
        acc_sh.at[pl.ds(sid * ROWS_PER_SUB, ROWS_PER_SUB)],
        out_hbm.at[cid, pl.ds(sid * ROWS_PER_SUB, ROWS_PER_SUB)],
    )


_BS = 2048


def _self_proj(x, w, b):
    D = w.shape[0]

    def body(x_ref, w_ref, b_ref, o_ref):
        o_ref[...] = lax.dot_general(
            x_ref[...], w_ref[...], (((1,), (1,)), ((), ())),
            preferred_element_type=jnp.float32,
        ) + b_ref[...]

    return pl.pallas_call(
        body,
        grid=(N_PAD // _BS,),
        in_specs=[
            pl.BlockSpec((_BS, F), lambda i: (i, 0)),
            pl.BlockSpec((D, F), lambda i: (0, 0)),
            pl.BlockSpec((1, D), lambda i: (0, 0)),
        ],
        out_specs=pl.BlockSpec((_BS, D), lambda i: (i, 0)),
        out_shape=jax.ShapeDtypeStruct((N_PAD, D), jnp.float32),
    )(x, w, b.reshape(1, D))


def _combine(z, w_neigh, aggp, degp):
    D = w_neigh.shape[0]

    def body(z_ref, w_ref, a_ref, d_ref, o_ref):
        deg = d_ref[0, :, 0:1] + d_ref[1, :, 0:1]
        inv = 1.0 / jnp.maximum(deg, 1.0)
        mean = (a_ref[0] + a_ref[1]) * inv
        mn = lax.dot_general(
            mean, w_ref[...], (((1,), (1,)), ((), ())),
            preferred_element_type=jnp.float32,
        )
        o_ref[...] = jax.nn.sigmoid(z_ref[...] + mn)

    return pl.pallas_call(
        body,
        grid=(N_PAD // _BS,),
        in_specs=[
            pl.BlockSpec((_BS, D), lambda i: (i, 0)),
            pl.BlockSpec((D, F), lambda i: (0, 0)),
            pl.BlockSpec((2, _BS, F), lambda i: (0, i, 0)),
            pl.BlockSpec((2, _BS, F), lambda i: (0, i, 0)),
        ],
        out_specs=pl.BlockSpec((_BS, D), lambda i: (i, 0)),
        out_shape=jax.ShapeDtypeStruct((N_PAD, D), jnp.float32),
    )(z, w_neigh, aggp, degp)


def kernel(inputs, W1_self, W1_neigh, b1, W2_self, W2_neigh, b2,
           W3_self, W3_neigh, b3, edge_index):
    x = jnp.pad(inputs, ((0, N_PAD - N_NODES), (0, 0)))
    pad2 = ((0, 0), (0, NCH * CH - EPW))
    srcm = jnp.pad(edge_index[0].reshape(NW, EPW), pad2,
                   constant_values=N_NODES).reshape(NW * NCH, CH)
    dstm = jnp.pad(edge_index[1].reshape(NW, EPW), pad2,
                   constant_values=N_NODES).reshape(NW * NCH, CH)

    degp = _deg_kernel(dstm)

    h = x
    for w_self, w_neigh, b in ((W1_self, W1_neigh, b1),
                               (W2_self, W2_neigh, b2),
                               (W3_self, W3_neigh, b3)):
        aggp = _agg(h, srcm, dstm)
        z = _self_proj(h, w_self, b)
        h = _combine(z, w_neigh, aggp, degp)

    return h[:N_NODES]


# speedup vs baseline: 3.7798x; 1.0016x over previous
import functools

import jax
import jax.numpy as jnp
from jax import lax
from jax.experimental import pallas as pl
from jax.experimental.pallas import tpu as pltpu
from jax.experimental.pallas import tpu_sc as plsc

N_NODES = 10000
N_PAD = 10240
N_EDGES = 320000
F = 128
NW = 32
CH = 64
EPW = N_EDGES // NW
NCH = 160
SUP = 8
NBUF = 4
ROWS_PER_SUB = N_PAD // 16

_MESH = plsc.VectorSubcoreMesh(core_axis_name="c", subcore_axis_name="s")


NGRP = NCH // SUP


@functools.partial(
    pl.kernel,
    out_type=jax.ShapeDtypeStruct((2, N_PAD, F), jnp.float32),
    mesh=_MESH,
    scratch_types=[
        pltpu.VMEM((SUP, CH), jnp.int32),
        pltpu.VMEM((SUP, CH), jnp.int32),
        pltpu.VMEM((SUP, CH), jnp.int32),
        pltpu.VMEM((SUP, CH), jnp.int32),
        pltpu.VMEM((CH, F), jnp.float32),
        pltpu.VMEM((CH, F), jnp.float32),
        pltpu.VMEM((CH, F), jnp.float32),
        pltpu.VMEM((CH, F), jnp.float32),
        pltpu.VMEM_SHARED((N_PAD, F), jnp.float32),
        pltpu.SemaphoreType.DMA,
        pltpu.SemaphoreType.DMA,
        pltpu.SemaphoreType.DMA,
        pltpu.SemaphoreType.DMA,
        pltpu.SemaphoreType.DMA,
    ],
)
def _agg(x_hbm, src_hbm, dst_hbm, out_hbm, srcA, dstA, srcB, dstB,
         buf0, buf1, buf2, buf3, acc_sh, gsem0, gsem1, gsem2, gsem3, isem):
    bufs = (buf0, buf1, buf2, buf3)
    gsems = (gsem0, gsem1, gsem2, gsem3)
    idxs = ((srcA, dstA), (srcB, dstB))
    cid = lax.axis_index("c")
    sid = lax.axis_index("s")
    wid = sid * 2 + cid

    @pl.loop(0, CH)
    def _(i):
        @pl.loop(0, F // 16)
        def _(k):
            buf0[i, pl.ds(k * 16, 16)] = jnp.zeros((16,), jnp.float32)

    @pl.loop(0, ROWS_PER_SUB // CH)
    def _(r):
        pltpu.sync_copy(buf0, acc_sh.at[pl.ds(sid * ROWS_PER_SUB + r * CH, CH)])

    plsc.subcore_barrier()

    def wait_gather(b):
        pltpu.make_async_copy(x_hbm.at[srcA.at[0]], bufs[b], gsems[b]).wait()

    def stage(s, par):
        base = wid * NCH + s * SUP
        pltpu.async_copy(src_hbm.at[pl.ds(base, SUP)], idxs[par][0], isem)
        pltpu.async_copy(dst_hbm.at[pl.ds(base, SUP)], idxs[par][1], isem)

    def wait_stage():
        pltpu.make_async_copy(src_hbm.at[pl.ds(0, SUP)], srcA, isem).wait()
        pltpu.make_async_copy(src_hbm.at[pl.ds(0, SUP)], srcB, isem).wait()

    stage(0, 0)
    wait_stage()
    for b in range(NBUF):
        pltpu.async_copy(x_hbm.at[srcA.at[b]], bufs[b], gsems[b])

    @pl.loop(0, NGRP // 2)
    def _(g2):
        for par in range(2):
            s = g2 * 2 + par
            cur_s, cur_d = idxs[par]
            nxt_s, _ = idxs[1 - par]

            @pl.when(s + 1 < NGRP)
            def _():
                stage(s + 1, 1 - par)

            for u in range(SUP):
                b = u % NBUF
                wait_gather(b)
                pltpu.sync_copy(bufs[b], acc_sh.at[cur_d.at[u]], add=True)
                if u == SUP - NBUF:
                    @pl.when(s + 1 < NGRP)
                    def _():
                        wait_stage()
                nxt_row = (cur_s.at[u + NBUF] if u + NBUF < SUP
                           else nxt_s.at[u + NBUF - SUP])

                @pl.when(s * SUP + u + NBUF < NCH)
                def _():
                    pltpu.async_copy(x_hbm.at[nxt_row], bufs[b], gsems[b])

    plsc.subcore_barrier()
    pltpu.sync_copy(
        acc_sh.at[pl.ds(sid * ROWS_PER_SUB, ROWS_PER_SUB)],
        out_hbm.at[cid, pl.ds(sid * ROWS_PER_SUB, ROWS_PER_SUB)],
    )


DCH = 128


@functools.partial(
    pl.kernel,
    out_type=jax.ShapeDtypeStruct((2, N_PAD, F), jnp.float32),
    mesh=_MESH,
    scratch_types=[
        pltpu.VMEM((NCH * CH // DCH, DCH), jnp.int32),
        pltpu.VMEM((DCH, F), jnp.float32),
        pltpu.VMEM_SHARED((N_PAD, F), jnp.float32),
        pltpu.SemaphoreType.DMA,
        pltpu.SemaphoreType.DMA,
    ],
)
def _deg_kernel(dst_hbm, out_hbm, dst_v, ones_v, acc_sh, ssem0, ssem1):
    ssems = (ssem0, ssem1)
    KCH = NCH * CH // DCH
    cid = lax.axis_index("c")
    sid = lax.axis_index("s")
    wid = sid * 2 + cid

    @pl.loop(0, DCH)
    def _(i):
        @pl.loop(0, F // 16)
        def _(k):
            ones_v[i, pl.ds(k * 16, 16)] = jnp.zeros((16,), jnp.float32)

    @pl.loop(0, ROWS_PER_SUB // DCH)
    def _(r):
        pltpu.sync_copy(ones_v, acc_sh.at[pl.ds(sid * ROWS_PER_SUB + r * DCH, DCH)])

    @pl.loop(0, DCH)
    def _(i):
        @pl.loop(0, F // 16)
        def _(k):
            ones_v[i, pl.ds(k * 16, 16)] = jnp.ones((16,), jnp.float32)

    plsc.subcore_barrier()
    pltpu.sync_copy(dst_hbm.at[pl.ds(wid * KCH, KCH)], dst_v)

    pltpu.async_copy(ones_v, acc_sh.at[dst_v.at[0]], ssem0, add=True)
    pltpu.async_copy(ones_v, acc_sh.at[dst_v.at[1]], ssem1, add=True)

    @pl.loop(0, KCH // 2)
    def _(t):
        for par in range(2):
            pltpu.make_async_copy(
                ones_v, acc_sh.at[dst_v.at[0]], ssems[par]).wait()

            @pl.when(t * 2 + par + 2 < KCH)
            def _():
                pltpu.async_copy(
                    ones_v, acc_sh.at[dst_v.at[t * 2 + par + 2]],
                    ssems[par], add=True)

    plsc.subcore_barrier()
    pltpu.sync_copy(
        acc_sh.at[pl.ds(sid * ROWS_PER_SUB, ROWS_PER_SUB)],
        out_hbm.at[cid, pl.ds(sid * ROWS_PER_SUB, ROWS_PER_SUB)],
    )


_BS = 2048


def _self_proj(x, w, b):
    D = w.shape[0]

    def body(x_ref, w_ref, b_ref, o_ref):
        o_ref[...] = lax.dot_general(
            x_ref[...], w_ref[...], (((1,), (1,)), ((), ())),
            preferred_element_type=jnp.float32,
        ) + b_ref[...]

    return pl.pallas_call(
        body,
        grid=(N_PAD // _BS,),
        in_specs=[
            pl.BlockSpec((_BS, F), lambda i: (i, 0)),
            pl.BlockSpec((D, F), lambda i: (0, 0)),
            pl.BlockSpec((1, D), lambda i: (0, 0)),
        ],
        out_specs=pl.BlockSpec((_BS, D), lambda i: (i, 0)),
        out_shape=jax.ShapeDtypeStruct((N_PAD, D), jnp.float32),
    )(x, w, b.reshape(1, D))


def _combine(z, w_neigh, aggp, degp):
    D = w_neigh.shape[0]

    def body(z_ref, w_ref, a_ref, d_ref, o_ref):
        deg = d_ref[0, :, 0:1] + d_ref[1, :, 0:1]
        inv = 1.0 / jnp.maximum(deg, 1.0)
        mean = (a_ref[0] + a_ref[1]) * inv
        mn = lax.dot_general(
            mean, w_ref[...], (((1,), (1,)), ((), ())),
            preferred_element_type=jnp.float32,
        )
        o_ref[...] = jax.nn.sigmoid(z_ref[...] + mn)

    return pl.pallas_call(
        body,
        grid=(N_PAD // _BS,),
        in_specs=[
            pl.BlockSpec((_BS, D), lambda i: (i, 0)),
            pl.BlockSpec((D, F), lambda i: (0, 0)),
            pl.BlockSpec((2, _BS, F), lambda i: (0, i, 0)),
            pl.BlockSpec((2, _BS, F), lambda i: (0, i, 0)),
        ],
        out_specs=pl.BlockSpec((_BS, D), lambda i: (i, 0)),
        out_shape=jax.ShapeDtypeStruct((N_PAD, D), jnp.float32),
    )(z, w_neigh, aggp, degp)


def kernel(inputs, W1_self, W1_neigh, b1, W2_self, W2_neigh, b2,
           W3_self, W3_neigh, b3, edge_index):
    x = jnp.pad(inputs, ((0, N_PAD - N_NODES), (0, 0)))
    pad2 = ((0, 0), (0, NCH * CH - EPW))
    srcm = jnp.pad(edge_index[0].reshape(NW, EPW), pad2,
                   constant_values=N_NODES).reshape(NW * NCH, CH)
    dstm = jnp.pad(edge_index[1].reshape(NW, EPW), pad2,
                   constant_values=N_NODES).reshape(NW * NCH, CH)

    degp = _deg_kernel(dstm.reshape(-1, 128))

    h = x
    for w_self, w_neigh, b in ((W1_self, W1_neigh, b1),
                               (W2_self, W2_neigh, b2),
                               (W3_self, W3_neigh, b3)):
        aggp = _agg(h, srcm, dstm)
        z = _self_proj(h, w_self, b)
        h = _combine(z, w_neigh, aggp, degp)

    return h[:N_NODES]
